# Initial kernel scaffold; baseline (speedup 1.0000x reference)
import jax
import jax.numpy as jnp
from jax.experimental import pallas as pl


def kernel(x, cv1_w, cv1_b, cv2_w, cv2_b, router_w, router_b, shared_w, shared_b, experts_w, experts_b):
    raise NotImplementedError("write your pallas kernel here")


# speedup vs baseline: 1.6624x; 1.6624x over previous
import functools

import jax
import jax.numpy as jnp
from jax.experimental import pallas as pl
from jax.experimental.pallas import tpu as pltpu


def _silu(v):
    return v * jax.nn.sigmoid(v)


def _c2f_moe_kernel(x_ref, cv1_w_ref, cv1_b_ref, cv2_w_ref, cv2_b_ref,
                    router_w_ref, router_b_ref, shared_w_ref, shared_b_ref,
                    experts_w_ref, experts_b_ref, out_ref):
    f32 = jnp.float32
    xb = x_ref[0]
    y = _silu(jnp.dot(cv1_w_ref[...], xb, preferred_element_type=f32)
              + cv1_b_ref[...])
    c = y.shape[0] // 2
    y2 = y[c:, :]

    hw = y2.shape[1]
    pooled = jnp.sum(y2, axis=1, keepdims=True) * (1.0 / hw)
    logits = (jnp.dot(router_w_ref[...], pooled, preferred_element_type=f32)
              + router_b_ref[...])
    lmax = jnp.max(logits)
    gate_w = 1.0 / jnp.sum(jnp.exp(logits - lmax))
    ids = jax.lax.broadcasted_iota(jnp.int32, logits.shape, 0)
    idx = jnp.min(jnp.where(logits >= lmax, ids, logits.shape[0]))

    ew = experts_w_ref[idx]
    eb = experts_b_ref[idx]
    shared = _silu(jnp.dot(shared_w_ref[...], y2, preferred_element_type=f32)
                   + shared_b_ref[...])
    routed = gate_w * _silu(jnp.dot(ew, y2, preferred_element_type=f32) + eb)
    moe = shared + routed

    w_y = cv2_w_ref[:, :2 * c]
    w_m = cv2_w_ref[:, 2 * c:3 * c] + cv2_w_ref[:, 3 * c:]
    out = _silu(jnp.dot(w_y, y, preferred_element_type=f32)
                + jnp.dot(w_m, moe, preferred_element_type=f32)
                + cv2_b_ref[...])
    out_ref[0] = out


@functools.partial(jax.jit, static_argnames=("interpret",))
def kernel(x, cv1_w, cv1_b, cv2_w, cv2_b, router_w, router_b,
           shared_w, shared_b, experts_w, experts_b, interpret=False):
    B, C1, H, W = x.shape
    HW = H * W
    O = cv2_w.shape[0]
    E, c, _ = experts_w.shape

    x3 = x.reshape(B, C1, HW)
    full = lambda a: pl.BlockSpec(a.shape, lambda i: (0,) * a.ndim)
    args = (
        x3,
        cv1_w, cv1_b.reshape(-1, 1),
        cv2_w, cv2_b.reshape(-1, 1),
        router_w, router_b.reshape(-1, 1),
        shared_w, shared_b.reshape(-1, 1),
        experts_w, experts_b.reshape(E, c, 1),
    )
    in_specs = [pl.BlockSpec((1, C1, HW), lambda i: (i, 0, 0))]
    in_specs += [full(a) for a in args[1:]]
    out = pl.pallas_call(
        _c2f_moe_kernel,
        grid=(B,),
        in_specs=in_specs,
        out_specs=pl.BlockSpec((1, O, HW), lambda i: (i, 0, 0)),
        out_shape=jax.ShapeDtypeStruct((B, O, HW), jnp.float32),
        compiler_params=pltpu.CompilerParams(
            dimension_semantics=("parallel",)),
        interpret=interpret,
    )(*args)
    return out.reshape(B, O, H, W)
